# baseline (speedup 1.0000x reference)
import functools

import jax
import jax.numpy as jnp
from jax import lax
from jax.experimental import pallas as pl
from jax.experimental.pallas import tpu as pltpu
from jax.experimental.pallas import tpu_sc as plsc

NUM_TYPES = 10000
D = 128
B = 16
L = 200
LP = 256
NW = 32
RPW = (B * L) // NW
EPW = RPW * LP
NCHUNK = EPW // 128
FIRE = 8


AMAIN = 9984
AROW = AMAIN + 128


def _sc_gather_g(ev_flat, a, a_tail):
    mesh = plsc.VectorSubcoreMesh(core_axis_name="c", subcore_axis_name="s")

    @functools.partial(
        pl.kernel,
        mesh=mesh,
        out_type=jax.ShapeDtypeStruct((NW, RPW, LP), jnp.float32),
        scratch_types=[
            pltpu.VMEM((LP,), jnp.int32),
            pltpu.VMEM((16,), jnp.int32),
            pltpu.VMEM((1, AROW), jnp.float32),
            pltpu.VMEM((RPW, LP), jnp.float32),
            pltpu.SemaphoreType.DMA,
        ],
        compiler_params=pltpu.CompilerParams(needs_layout_passes=False),
    )
    def k(ev_hbm, a_hbm, atail_hbm, out_hbm, ev_v, rid_v, row_v, rows_v, sem):
        wid = lax.axis_index("s") * 2 + lax.axis_index("c")
        b = wid // 2
        i0 = (wid % 2) * RPW

        zeros16 = jnp.zeros((16,), jnp.int32)
        for cc in range(L // 16, LP // 16):
            ev_v[pl.ds(cc * 16, 16)] = zeros16
        pltpu.sync_copy(ev_hbm.at[pl.ds(b * L, L)], ev_v.at[pl.ds(0, L)])

        lane = lax.broadcasted_iota(jnp.int32, (16,), 0)

        def do_row(r, carry):
            i = i0 + r
            chunk = ev_v[pl.ds((i // 16) * 16, 16)]
            rid = jnp.sum(jnp.where(lane == i % 16, chunk, 0))
            rid_v[...] = jnp.full((16,), rid, jnp.int32)
            ridx = rid_v.at[pl.ds(0, 1)]
            cp1 = pltpu.async_copy(a_hbm.at[ridx, pl.ds(0, AMAIN)],
                                   row_v.at[:, pl.ds(0, AMAIN)], sem)
            cp2 = pltpu.async_copy(atail_hbm.at[ridx],
                                   row_v.at[:, pl.ds(AMAIN, 128)], sem)
            cp1.wait()
            cp2.wait()
            for cc in range(LP // 16):
                vals = plsc.load_gather(
                    row_v, [zeros16, ev_v[pl.ds(cc * 16, 16)]])
                rows_v[r, pl.ds(cc * 16, 16)] = vals
            return carry

        lax.fori_loop(0, RPW, do_row, 0)

        pltpu.sync_copy(rows_v, out_hbm.at[wid])

    return k(ev_flat, a, a_tail)


def _tc_body(gp_ref, x_ref, w0_ref, b0_ref, w1_ref, b1_ref, out_ref):
    gp = gp_ref[0]
    ri = lax.broadcasted_iota(jnp.int32, (L, LP), 0)
    ci = lax.broadcasted_iota(jnp.int32, (L, LP), 1)
    diag = jnp.sum(jnp.where(ri == ci, gp, 0.0), axis=0, keepdims=True)
    adj = gp + diag
    x = x_ref[0]
    zpad = jnp.zeros((LP - L, D), jnp.float32)
    for w_ref, bias_ref in ((w0_ref, b0_ref), (w1_ref, b1_ref)):
        xp = jnp.concatenate([x, zpad], axis=0)
        h = jnp.dot(adj, xp, preferred_element_type=jnp.float32)
        hw = jnp.dot(h, w_ref[...], preferred_element_type=jnp.float32)
        x = jnp.maximum(hw + bias_ref[...], 0.0) + x
    out_ref[...] = jnp.mean(x, axis=0, keepdims=True)[None]


def _tc_encoder(gp, x, w0, b0, w1, b1):
    return pl.pallas_call(
        _tc_body,
        grid=(B,),
        in_specs=[
            pl.BlockSpec((1, L, LP), lambda i: (i, 0, 0)),
            pl.BlockSpec((1, L, D), lambda i: (i, 0, 0)),
            pl.BlockSpec((D, D), lambda i: (0, 0)),
            pl.BlockSpec((1, D), lambda i: (0, 0)),
            pl.BlockSpec((D, D), lambda i: (0, 0)),
            pl.BlockSpec((1, D), lambda i: (0, 0)),
        ],
        out_specs=pl.BlockSpec((1, 1, D), lambda i: (i, 0, 0)),
        out_shape=jax.ShapeDtypeStruct((B, 1, D), jnp.float32),
    )(gp, x, w0, b0, w1, b1).reshape(B, D)


@jax.jit
def kernel(event_type, enc_output, slf_attn_mask, non_pad_mask, ui_adj,
           W0, b0, W1, b1):
    ev = (event_type - 1).astype(jnp.int32).reshape(B * L)
    a_tail = jnp.pad(ui_adj[:, AMAIN:], ((0, 0), (0, 128 - (NUM_TYPES - AMAIN))))
    gp = _sc_gather_g(ev, ui_adj, a_tail).reshape(B, L, LP)
    eye = jnp.eye(W0.shape[0], dtype=jnp.float32)
    wbd0 = (eye[:, None, :, None] * W0[:, :, None, :]).reshape(D, D)
    wbd1 = (eye[:, None, :, None] * W1[:, :, None, :]).reshape(D, D)
    return _tc_encoder(gp, enc_output, wbd0, b0.reshape(1, D),
                       wbd1, b1.reshape(1, D))


# speedup vs baseline: 1.5633x; 1.5633x over previous
import functools

import jax
import jax.numpy as jnp
from jax import lax
from jax.experimental import pallas as pl
from jax.experimental.pallas import tpu as pltpu
from jax.experimental.pallas import tpu_sc as plsc

NUM_TYPES = 10000
D = 128
B = 16
L = 200
LP = 256
NW = 32
RPW = (B * L) // NW
EPW = RPW * LP
NCHUNK = EPW // 128
FIRE = 8


AMAIN = 9984
AROW = AMAIN + 128


def _sc_gather_g(ev_flat, a, a_tail):
    mesh = plsc.VectorSubcoreMesh(core_axis_name="c", subcore_axis_name="s")

    NBUF = 4
    NT = 112

    @functools.partial(
        pl.kernel,
        mesh=mesh,
        out_type=jax.ShapeDtypeStruct((NW, RPW, LP), jnp.float32),
        scratch_types=[
            pltpu.VMEM((LP,), jnp.int32),
            pltpu.VMEM((NT,), jnp.int32),
            pltpu.VMEM((NT, 128), jnp.float32),
            pltpu.VMEM((NBUF, 16), jnp.int32),
            pltpu.VMEM((1, AROW), jnp.float32),
            pltpu.VMEM((1, AROW), jnp.float32),
            pltpu.VMEM((1, AROW), jnp.float32),
            pltpu.VMEM((1, AROW), jnp.float32),
            pltpu.VMEM((RPW, LP), jnp.float32),
            pltpu.SemaphoreType.DMA,
            pltpu.SemaphoreType.DMA,
            pltpu.SemaphoreType.DMA,
            pltpu.SemaphoreType.DMA,
            pltpu.SemaphoreType.DMA,
        ],
        compiler_params=pltpu.CompilerParams(needs_layout_passes=False),
    )
    def k(ev_hbm, a_hbm, atail_hbm, out_hbm,
          ev_v, rids_v, tail_v, rid_v, ring0, ring1, ring2, ring3, rows_v,
          sem_t, s0, s1, s2, s3):
        sems = (s0, s1, s2, s3)
        rings = (ring0, ring1, ring2, ring3)
        wid = lax.axis_index("s") * 2 + lax.axis_index("c")
        b = wid // 2
        i0 = (wid % 2) * RPW

        zeros16 = jnp.zeros((16,), jnp.int32)
        for cc in range(L // 16, LP // 16):
            ev_v[pl.ds(cc * 16, 16)] = zeros16
        pltpu.sync_copy(ev_hbm.at[pl.ds(b * L, L)], ev_v.at[pl.ds(0, L)])

        for cc in range(NT // 16):
            rids_v[pl.ds(cc * 16, 16)] = ev_v[pl.ds(i0 + cc * 16, 16)]
        cp_tail = pltpu.async_copy(atail_hbm.at[rids_v], tail_v, sem_t)

        lane = lax.broadcasted_iota(jnp.int32, (16,), 0)

        def fire(r, t):
            i = i0 + r
            chunk = ev_v[pl.ds((i // 16) * 16, 16)]
            rid = jnp.sum(jnp.where(lane == i % 16, chunk, 0))
            rid_v[t, pl.ds(0, 16)] = jnp.full((16,), rid, jnp.int32)
            pltpu.async_copy(
                a_hbm.at[rid_v.at[t, pl.ds(0, 1)], pl.ds(0, AMAIN)],
                rings[t].at[:, pl.ds(0, AMAIN)], sems[t])

        def wait_buf(t):
            pltpu.make_async_copy(
                a_hbm.at[pl.ds(0, 1), pl.ds(0, AMAIN)],
                rings[t].at[:, pl.ds(0, AMAIN)], sems[t]).wait()

        for t in range(NBUF - 1):
            fire(t, t)
        cp_tail.wait()

        def step(q, carry):
            for t in range(NBUF):
                r = q * NBUF + t
                fire((r + NBUF - 1) % RPW, (t + NBUF - 1) % NBUF)
                wait_buf(t)
                for cc in range(128 // 16):
                    rings[t][0, pl.ds(AMAIN + cc * 16, 16)] = (
                        tail_v[r, pl.ds(cc * 16, 16)])
                for cc in range(LP // 16):
                    vals = plsc.load_gather(
                        rings[t], [zeros16, ev_v[pl.ds(cc * 16, 16)]])
                    rows_v[r, pl.ds(cc * 16, 16)] = vals
            return carry

        lax.fori_loop(0, RPW // NBUF, step, 0)
        for t in range(NBUF - 1):
            wait_buf(t)

        pltpu.sync_copy(rows_v, out_hbm.at[wid])

    return k(ev_flat, a, a_tail)


def _tc_body(gp_ref, x_ref, w0_ref, b0_ref, w1_ref, b1_ref, out_ref):
    gp = gp_ref[0]
    ri = lax.broadcasted_iota(jnp.int32, (L, LP), 0)
    ci = lax.broadcasted_iota(jnp.int32, (L, LP), 1)
    diag = jnp.sum(jnp.where(ri == ci, gp, 0.0), axis=0, keepdims=True)
    adj = gp + diag
    x = x_ref[0]
    zpad = jnp.zeros((LP - L, D), jnp.float32)
    for w_ref, bias_ref in ((w0_ref, b0_ref), (w1_ref, b1_ref)):
        xp = jnp.concatenate([x, zpad], axis=0)
        h = jnp.dot(adj, xp, preferred_element_type=jnp.float32)
        hw = jnp.dot(h, w_ref[...], preferred_element_type=jnp.float32)
        x = jnp.maximum(hw + bias_ref[...], 0.0) + x
    out_ref[...] = jnp.mean(x, axis=0, keepdims=True)[None]


def _tc_encoder(gp, x, w0, b0, w1, b1):
    return pl.pallas_call(
        _tc_body,
        grid=(B,),
        in_specs=[
            pl.BlockSpec((1, L, LP), lambda i: (i, 0, 0)),
            pl.BlockSpec((1, L, D), lambda i: (i, 0, 0)),
            pl.BlockSpec((D, D), lambda i: (0, 0)),
            pl.BlockSpec((1, D), lambda i: (0, 0)),
            pl.BlockSpec((D, D), lambda i: (0, 0)),
            pl.BlockSpec((1, D), lambda i: (0, 0)),
        ],
        out_specs=pl.BlockSpec((1, 1, D), lambda i: (i, 0, 0)),
        out_shape=jax.ShapeDtypeStruct((B, 1, D), jnp.float32),
    )(gp, x, w0, b0, w1, b1).reshape(B, D)


@jax.jit
def kernel(event_type, enc_output, slf_attn_mask, non_pad_mask, ui_adj,
           W0, b0, W1, b1):
    ev = (event_type - 1).astype(jnp.int32).reshape(B * L)
    a_tail = jnp.pad(ui_adj[:, AMAIN:], ((0, 0), (0, 128 - (NUM_TYPES - AMAIN))))
    gp = _sc_gather_g(ev, ui_adj, a_tail).reshape(B, L, LP)
    eye = jnp.eye(W0.shape[0], dtype=jnp.float32)
    wbd0 = (eye[:, None, :, None] * W0[:, :, None, :]).reshape(D, D)
    wbd1 = (eye[:, None, :, None] * W1[:, :, None, :]).reshape(D, D)
    return _tc_encoder(gp, enc_output, wbd0, b0.reshape(1, D),
                       wbd1, b1.reshape(1, D))


# speedup vs baseline: 1.6730x; 1.0702x over previous
import functools

import jax
import jax.numpy as jnp
from jax import lax
from jax.experimental import pallas as pl
from jax.experimental.pallas import tpu as pltpu
from jax.experimental.pallas import tpu_sc as plsc

NUM_TYPES = 10000
D = 128
B = 16
L = 200
LP = 256
NW = 32
RPW = (B * L) // NW
EPW = RPW * LP
NCHUNK = EPW // 128
FIRE = 8


AMAIN = 9984
AROW = AMAIN + 128


def _sc_gather_g(ev_flat, a, a_tail):
    mesh = plsc.VectorSubcoreMesh(core_axis_name="c", subcore_axis_name="s")

    NBUF = 8
    NT = 112

    @functools.partial(
        pl.kernel,
        mesh=mesh,
        out_type=jax.ShapeDtypeStruct((NW, RPW, LP), jnp.float32),
        scratch_types=[
            pltpu.VMEM((LP,), jnp.int32),
            pltpu.VMEM((NT,), jnp.int32),
            pltpu.VMEM((NT, 128), jnp.float32),
            pltpu.VMEM((NBUF, 16), jnp.int32),
        ]
        + [pltpu.VMEM((1, AROW), jnp.float32) for _ in range(NBUF)]
        + [pltpu.VMEM((RPW, LP), jnp.float32)]
        + [pltpu.SemaphoreType.DMA for _ in range(NBUF + 1)],
        compiler_params=pltpu.CompilerParams(needs_layout_passes=False),
    )
    def k(ev_hbm, a_hbm, atail_hbm, out_hbm,
          ev_v, rids_v, tail_v, rid_v, *rest):
        rings = rest[:NBUF]
        rows_v = rest[NBUF]
        sem_t = rest[NBUF + 1]
        sems = rest[NBUF + 2:]
        wid = lax.axis_index("s") * 2 + lax.axis_index("c")
        b = wid // 2
        i0 = (wid % 2) * RPW

        zeros16 = jnp.zeros((16,), jnp.int32)
        for cc in range(L // 16, LP // 16):
            ev_v[pl.ds(cc * 16, 16)] = zeros16
        pltpu.sync_copy(ev_hbm.at[pl.ds(b * L, L)], ev_v.at[pl.ds(0, L)])

        for cc in range(NT // 16):
            rids_v[pl.ds(cc * 16, 16)] = ev_v[pl.ds(i0 + cc * 16, 16)]
        cp_tail = pltpu.async_copy(atail_hbm.at[rids_v], tail_v, sem_t)

        lane = lax.broadcasted_iota(jnp.int32, (16,), 0)

        def fire(r, t):
            i = i0 + r
            chunk = ev_v[pl.ds((i // 16) * 16, 16)]
            rid = jnp.sum(jnp.where(lane == i % 16, chunk, 0))
            rid_v[t, pl.ds(0, 16)] = jnp.full((16,), rid, jnp.int32)
            pltpu.async_copy(
                a_hbm.at[rid_v.at[t, pl.ds(0, 1)], pl.ds(0, AMAIN)],
                rings[t].at[:, pl.ds(0, AMAIN)], sems[t])

        def wait_buf(t):
            pltpu.make_async_copy(
                a_hbm.at[pl.ds(0, 1), pl.ds(0, AMAIN)],
                rings[t].at[:, pl.ds(0, AMAIN)], sems[t]).wait()

        for t in range(NBUF - 1):
            fire(t, t)
        cp_tail.wait()

        def step(q, carry):
            for t in range(NBUF):
                r = q * NBUF + t
                fire((r + NBUF - 1) % RPW, (t + NBUF - 1) % NBUF)
                wait_buf(t)
                for cc in range(128 // 16):
                    rings[t][0, pl.ds(AMAIN + cc * 16, 16)] = (
                        tail_v[r, pl.ds(cc * 16, 16)])
                for cc in range(LP // 16):
                    vals = plsc.load_gather(
                        rings[t], [zeros16, ev_v[pl.ds(cc * 16, 16)]])
                    rows_v[r, pl.ds(cc * 16, 16)] = vals
            return carry

        lax.fori_loop(0, RPW // NBUF, step, 0)
        for t in range(NBUF - 1):
            wait_buf(t)

        pltpu.sync_copy(rows_v, out_hbm.at[wid])

    return k(ev_flat, a, a_tail)


def _tc_body(gp_ref, x_ref, w0_ref, b0_ref, w1_ref, b1_ref, out_ref):
    gp = gp_ref[0]
    ri = lax.broadcasted_iota(jnp.int32, (L, LP), 0)
    ci = lax.broadcasted_iota(jnp.int32, (L, LP), 1)
    diag = jnp.sum(jnp.where(ri == ci, gp, 0.0), axis=0, keepdims=True)
    adj = gp + diag
    x = x_ref[0]
    zpad = jnp.zeros((LP - L, D), jnp.float32)
    for w_ref, bias_ref in ((w0_ref, b0_ref), (w1_ref, b1_ref)):
        xp = jnp.concatenate([x, zpad], axis=0)
        h = jnp.dot(adj, xp, preferred_element_type=jnp.float32)
        hw = jnp.dot(h, w_ref[...], preferred_element_type=jnp.float32)
        x = jnp.maximum(hw + bias_ref[...], 0.0) + x
    out_ref[...] = jnp.mean(x, axis=0, keepdims=True)[None]


def _tc_encoder(gp, x, w0, b0, w1, b1):
    return pl.pallas_call(
        _tc_body,
        grid=(B,),
        in_specs=[
            pl.BlockSpec((1, L, LP), lambda i: (i, 0, 0)),
            pl.BlockSpec((1, L, D), lambda i: (i, 0, 0)),
            pl.BlockSpec((D, D), lambda i: (0, 0)),
            pl.BlockSpec((1, D), lambda i: (0, 0)),
            pl.BlockSpec((D, D), lambda i: (0, 0)),
            pl.BlockSpec((1, D), lambda i: (0, 0)),
        ],
        out_specs=pl.BlockSpec((1, 1, D), lambda i: (i, 0, 0)),
        out_shape=jax.ShapeDtypeStruct((B, 1, D), jnp.float32),
    )(gp, x, w0, b0, w1, b1).reshape(B, D)


@jax.jit
def kernel(event_type, enc_output, slf_attn_mask, non_pad_mask, ui_adj,
           W0, b0, W1, b1):
    ev = (event_type - 1).astype(jnp.int32).reshape(B * L)
    a_tail = jnp.pad(ui_adj[:, AMAIN:], ((0, 0), (0, 128 - (NUM_TYPES - AMAIN))))
    gp = _sc_gather_g(ev, ui_adj, a_tail).reshape(B, L, LP)
    eye = jnp.eye(W0.shape[0], dtype=jnp.float32)
    wbd0 = (eye[:, None, :, None] * W0[:, :, None, :]).reshape(D, D)
    wbd1 = (eye[:, None, :, None] * W1[:, :, None, :]).reshape(D, D)
    return _tc_encoder(gp, enc_output, wbd0, b0.reshape(1, D),
                       wbd1, b1.reshape(1, D))
